# Initial kernel scaffold; baseline (speedup 1.0000x reference)
import jax
import jax.numpy as jnp
from jax.experimental import pallas as pl


def kernel(x_0, node_idx, edge_idx, W2):
    raise NotImplementedError("write your pallas kernel here")


# speedup vs baseline: 3.2654x; 3.2654x over previous
import functools

import jax
import jax.numpy as jnp
from jax import lax
from jax.experimental import pallas as pl
from jax.experimental.pallas import tpu as pltpu
from jax.experimental.pallas import tpu_sc as plsc

N_NODES = 10000
N_HEDGES = 20000
NNZ = 320000
D = 128

NC = 2
NT = 16
LANES = 16

HALF_E = N_HEDGES // NC
ACC_E_ROWS = HALF_E + 16
CHUNK = 80

WROWS = 624
TAIL = N_NODES - NT * WROWS

_MESH = plsc.VectorSubcoreMesh(core_axis_name="c", subcore_axis_name="s")


def _x1_body(x0_hbm, nidx_hbm, eidx_hbm, zero_hbm, x1_hbm,
             acc, nidx_v, eidx_v, lidx_v, rows_v, sem):
    c = lax.axis_index("c")
    s = lax.axis_index("s")
    base_e = c * HALF_E

    pltpu.sync_copy(zero_hbm, acc.at[pl.ds(s * WROWS, WROWS)])

    @pl.when(s == 0)
    def _():
        pltpu.sync_copy(zero_hbm.at[pl.ds(0, TAIL)],
                        acc.at[pl.ds(NT * WROWS, TAIL)])

    plsc.subcore_barrier()

    per_tile = NNZ // NT
    n_chunks = per_tile // CHUNK

    def chunk_body(i, carry):
        off = s * per_tile + i * CHUNK
        pltpu.sync_copy(nidx_hbm.at[pl.ds(off, CHUNK)], nidx_v)
        pltpu.sync_copy(eidx_hbm.at[pl.ds(off, CHUNK)], eidx_v)
        pltpu.async_copy(x0_hbm.at[nidx_v], rows_v, sem).wait()
        for j in range(CHUNK // LANES):
            e = eidx_v[pl.ds(j * LANES, LANES)]
            l = e - base_e
            ok = (l >= 0) & (l < HALF_E)
            lidx_v[pl.ds(j * LANES, LANES)] = jnp.where(ok, l, HALF_E)
        pltpu.sync_copy(rows_v, acc.at[lidx_v], add=True)
        return carry

    lax.fori_loop(0, n_chunks, chunk_body, 0)
    plsc.subcore_barrier()

    pltpu.sync_copy(acc.at[pl.ds(s * WROWS, WROWS)],
                    x1_hbm.at[pl.ds(base_e + s * WROWS, WROWS)])

    @pl.when(s == 0)
    def _():
        pltpu.sync_copy(acc.at[pl.ds(NT * WROWS, TAIL)],
                        x1_hbm.at[pl.ds(base_e + NT * WROWS, TAIL)])


@functools.partial(
    pl.kernel,
    mesh=_MESH,
    out_type=jax.ShapeDtypeStruct((N_HEDGES, D), jnp.float32),
    scratch_types=[
        pltpu.VMEM_SHARED((ACC_E_ROWS, D), jnp.float32),
        pltpu.VMEM((CHUNK,), jnp.int32),
        pltpu.VMEM((CHUNK,), jnp.int32),
        pltpu.VMEM((CHUNK,), jnp.int32),
        pltpu.VMEM((CHUNK, D), jnp.float32),
        pltpu.SemaphoreType.DMA,
    ],
)
def _x1_kernel(x0_hbm, nidx_hbm, eidx_hbm, zero_hbm, x1_hbm,
               acc, nidx_v, eidx_v, lidx_v, rows_v, sem):
    _x1_body(x0_hbm, nidx_hbm, eidx_hbm, zero_hbm, x1_hbm,
             acc, nidx_v, eidx_v, lidx_v, rows_v, sem)


def _pre_body(x1_hbm, nidx_hbm, eidx_hbm, zero_hbm, pre_hbm,
              acc, nidx_v, eidx_v, rows_v, sem):
    c = lax.axis_index("c")
    s = lax.axis_index("s")

    pltpu.sync_copy(zero_hbm, acc.at[pl.ds(s * WROWS, WROWS)])

    @pl.when(s == 0)
    def _():
        pltpu.sync_copy(zero_hbm.at[pl.ds(0, TAIL)],
                        acc.at[pl.ds(NT * WROWS, TAIL)])

    plsc.subcore_barrier()

    per_tile = NNZ // (NC * NT)
    n_chunks = per_tile // CHUNK

    def chunk_body(i, carry):
        off = (c * NT + s) * per_tile + i * CHUNK
        pltpu.sync_copy(nidx_hbm.at[pl.ds(off, CHUNK)], nidx_v)
        pltpu.sync_copy(eidx_hbm.at[pl.ds(off, CHUNK)], eidx_v)
        pltpu.async_copy(x1_hbm.at[eidx_v], rows_v, sem).wait()
        pltpu.sync_copy(rows_v, acc.at[nidx_v], add=True)
        return carry

    lax.fori_loop(0, n_chunks, chunk_body, 0)
    plsc.subcore_barrier()

    pltpu.sync_copy(acc.at[pl.ds(s * WROWS, WROWS)],
                    pre_hbm.at[c, pl.ds(s * WROWS, WROWS)])

    @pl.when(s == 0)
    def _():
        pltpu.sync_copy(acc.at[pl.ds(NT * WROWS, TAIL)],
                        pre_hbm.at[c, pl.ds(NT * WROWS, TAIL)])


@functools.partial(
    pl.kernel,
    mesh=_MESH,
    out_type=jax.ShapeDtypeStruct((NC, N_NODES, D), jnp.float32),
    scratch_types=[
        pltpu.VMEM_SHARED((N_NODES, D), jnp.float32),
        pltpu.VMEM((CHUNK,), jnp.int32),
        pltpu.VMEM((CHUNK,), jnp.int32),
        pltpu.VMEM((CHUNK, D), jnp.float32),
        pltpu.SemaphoreType.DMA,
    ],
)
def _pre_kernel(x1_hbm, nidx_hbm, eidx_hbm, zero_hbm, pre_hbm,
                acc, nidx_v, eidx_v, rows_v, sem):
    _pre_body(x1_hbm, nidx_hbm, eidx_hbm, zero_hbm, pre_hbm,
              acc, nidx_v, eidx_v, rows_v, sem)


MM_BLK = 1000


def _mm_body(p0_ref, p1_ref, w_ref, o_ref):
    o_ref[...] = jnp.dot(p0_ref[...] + p1_ref[...], w_ref[...],
                         preferred_element_type=jnp.float32)


def _matmul(p0, p1, w):
    return pl.pallas_call(
        _mm_body,
        grid=(N_NODES // MM_BLK,),
        in_specs=[
            pl.BlockSpec((MM_BLK, D), lambda i: (i, 0)),
            pl.BlockSpec((MM_BLK, D), lambda i: (i, 0)),
            pl.BlockSpec((D, D), lambda i: (0, 0)),
        ],
        out_specs=pl.BlockSpec((MM_BLK, D), lambda i: (i, 0)),
        out_shape=jax.ShapeDtypeStruct((N_NODES, D), jnp.float32),
    )(p0, p1, w)


def kernel(x_0, node_idx, edge_idx, W2):
    zero_block = jnp.zeros((WROWS, D), jnp.float32)
    x_1 = _x1_kernel(x_0, node_idx, edge_idx, zero_block)
    pre = _pre_kernel(x_1, node_idx, edge_idx, zero_block)
    x_0_out = _matmul(pre[0], pre[1], W2)
    return (x_0_out, x_1)


# speedup vs baseline: 6.2605x; 1.9172x over previous
import functools

import jax
import jax.numpy as jnp
from jax import lax
from jax.experimental import pallas as pl
from jax.experimental.pallas import tpu as pltpu
from jax.experimental.pallas import tpu_sc as plsc

N_NODES = 10000
N_HEDGES = 20000
NNZ = 320000
D = 128

NC = 2
NT = 16
LANES = 16

HALF_E = N_HEDGES // NC
ACC_E_ROWS = HALF_E + 16
CHUNK = 80
PK = 2 * CHUNK
NROWS = NNZ // CHUNK

EZ = 624
EZ_TAIL = HALF_E - NT * EZ
NZ = 624
NZ_TAIL = N_NODES - NT * NZ

K1_CHUNKS = NROWS // NT
K2_CHUNKS = NROWS // (NC * NT)

_MESH = plsc.VectorSubcoreMesh(core_axis_name="c", subcore_axis_name="s")


def _sweep(src_hbm, pidx_hbm, zero_hbm, out_hbm,
           acc, idx0, idx1, lidx, rows0, rows1,
           gsem0, gsem1, isem0, isem1,
           n_chunks, row_base, gather_off, scatter_off, remap_base,
           zrows, ztail, out_row):
    s = lax.axis_index("s")

    pltpu.sync_copy(zero_hbm.at[pl.ds(0, zrows)],
                    acc.at[pl.ds(s * zrows, zrows)])

    @pl.when(s == 0)
    def _():
        pltpu.sync_copy(zero_hbm.at[pl.ds(0, ztail)],
                        acc.at[pl.ds(NT * zrows, ztail)])

    plsc.subcore_barrier()

    idxb = (idx0, idx1)
    rows = (rows0, rows1)
    gsem = (gsem0, gsem1)
    isem = (isem0, isem1)

    def start_idx(i, b):
        pltpu.async_copy(pidx_hbm.at[row_base + i], idxb[b], isem[b])

    def wait_idx(b):
        pltpu.make_async_copy(pidx_hbm.at[0], idxb[b], isem[b]).wait()

    def start_gather(b):
        pltpu.async_copy(src_hbm.at[idxb[b].at[pl.ds(gather_off, CHUNK)]],
                         rows[b], gsem[b])

    def wait_gather(b):
        pltpu.make_async_copy(src_hbm.at[pl.ds(0, CHUNK)], rows[b],
                              gsem[b]).wait()

    start_idx(0, 0)
    wait_idx(0)
    start_gather(0)
    start_idx(1, 1)

    def chunk(i, b):
        nb = 1 - b
        wait_gather(b)

        @pl.when(i + 1 < n_chunks)
        def _():
            wait_idx(nb)
            start_gather(nb)

        for j in range(CHUNK // LANES):
            e = idxb[b][pl.ds(scatter_off + j * LANES, LANES)]
            if remap_base is not None:
                l = e - remap_base
                ok = (l >= 0) & (l < HALF_E)
                e = jnp.where(ok, l, HALF_E)
            lidx[pl.ds(j * LANES, LANES)] = e

        @pl.when(i + 2 < n_chunks)
        def _():
            start_idx(i + 2, b)

        pltpu.sync_copy(rows[b], acc.at[lidx], add=True)

    def pair(p, carry):
        i = p * 2
        chunk(i, 0)

        @pl.when(i + 1 < n_chunks)
        def _():
            chunk(i + 1, 1)

        return carry

    lax.fori_loop(0, (n_chunks + 1) // 2, pair, 0)
    plsc.subcore_barrier()

    pltpu.sync_copy(acc.at[pl.ds(s * zrows, zrows)],
                    out_hbm.at[out_row, pl.ds(s * zrows, zrows)])

    @pl.when(s == 0)
    def _():
        pltpu.sync_copy(acc.at[pl.ds(NT * zrows, ztail)],
                        out_hbm.at[out_row, pl.ds(NT * zrows, ztail)])


def _sc_scratch(n_acc_rows):
    return [
        pltpu.VMEM_SHARED((n_acc_rows, D), jnp.float32),
        pltpu.VMEM((PK,), jnp.int32),
        pltpu.VMEM((PK,), jnp.int32),
        pltpu.VMEM((CHUNK,), jnp.int32),
        pltpu.VMEM((CHUNK, D), jnp.float32),
        pltpu.VMEM((CHUNK, D), jnp.float32),
        pltpu.SemaphoreType.DMA,
        pltpu.SemaphoreType.DMA,
        pltpu.SemaphoreType.DMA,
        pltpu.SemaphoreType.DMA,
    ]


@functools.partial(
    pl.kernel,
    mesh=_MESH,
    out_type=jax.ShapeDtypeStruct((NC, HALF_E, D), jnp.float32),
    scratch_types=_sc_scratch(ACC_E_ROWS),
)
def _x1_kernel(x0_hbm, pidx_hbm, zero_hbm, x1h_hbm,
               acc, idx0, idx1, lidx, rows0, rows1, gs0, gs1, is0, is1):
    c = lax.axis_index("c")
    s = lax.axis_index("s")
    _sweep(x0_hbm, pidx_hbm, zero_hbm, x1h_hbm,
           acc, idx0, idx1, lidx, rows0, rows1, gs0, gs1, is0, is1,
           n_chunks=K1_CHUNKS, row_base=s * K1_CHUNKS,
           gather_off=0, scatter_off=CHUNK, remap_base=c * HALF_E,
           zrows=EZ, ztail=EZ_TAIL, out_row=c)


@functools.partial(
    pl.kernel,
    mesh=_MESH,
    out_type=jax.ShapeDtypeStruct((NC, N_NODES, D), jnp.float32),
    scratch_types=_sc_scratch(N_NODES),
)
def _pre_kernel(x1_hbm, pidx_hbm, zero_hbm, pre_hbm,
                acc, idx0, idx1, lidx, rows0, rows1, gs0, gs1, is0, is1):
    c = lax.axis_index("c")
    s = lax.axis_index("s")
    _sweep(x1_hbm, pidx_hbm, zero_hbm, pre_hbm,
           acc, idx0, idx1, lidx, rows0, rows1, gs0, gs1, is0, is1,
           n_chunks=K2_CHUNKS, row_base=(c * NT + s) * K2_CHUNKS,
           gather_off=CHUNK, scatter_off=0, remap_base=None,
           zrows=NZ, ztail=NZ_TAIL, out_row=c)


MM_BLK = 1000


def _mm_body(p0_ref, p1_ref, w_ref, o_ref):
    o_ref[...] = jnp.dot(p0_ref[...] + p1_ref[...], w_ref[...],
                         preferred_element_type=jnp.float32)


def _matmul(p0, p1, w):
    return pl.pallas_call(
        _mm_body,
        grid=(N_NODES // MM_BLK,),
        in_specs=[
            pl.BlockSpec((MM_BLK, D), lambda i: (i, 0)),
            pl.BlockSpec((MM_BLK, D), lambda i: (i, 0)),
            pl.BlockSpec((D, D), lambda i: (0, 0)),
        ],
        out_specs=pl.BlockSpec((MM_BLK, D), lambda i: (i, 0)),
        out_shape=jax.ShapeDtypeStruct((N_NODES, D), jnp.float32),
    )(p0, p1, w)


def kernel(x_0, node_idx, edge_idx, W2):
    pidx = jnp.concatenate([node_idx.reshape(NROWS, CHUNK),
                            edge_idx.reshape(NROWS, CHUNK)], axis=1)
    zero_block = jnp.zeros((EZ, D), jnp.float32)
    x1h = _x1_kernel(x_0, pidx, zero_block)
    x_1 = x1h.reshape(N_HEDGES, D)
    pre = _pre_kernel(x_1, pidx, zero_block)
    x_0_out = _matmul(pre[0], pre[1], W2)
    return (x_0_out, x_1)


# speedup vs baseline: 6.2643x; 1.0006x over previous
import functools

import jax
import jax.numpy as jnp
from jax import lax
from jax.experimental import pallas as pl
from jax.experimental.pallas import tpu as pltpu
from jax.experimental.pallas import tpu_sc as plsc

N_NODES = 10000
N_HEDGES = 20000
NNZ = 320000
D = 128

NC = 2
NT = 16
LANES = 16

HALF_E = N_HEDGES // NC
ACC_E_ROWS = HALF_E + 16
CHUNK = 80
PK = 2 * CHUNK
NROWS = NNZ // CHUNK

EZ = 624
EZ_TAIL = HALF_E - NT * EZ
NZ = 624
NZ_TAIL = N_NODES - NT * NZ

K1_CHUNKS = NROWS // NT
K2_CHUNKS = NROWS // (NC * NT)

_MESH = plsc.VectorSubcoreMesh(core_axis_name="c", subcore_axis_name="s")


def _sweep(src_hbm, pidx_hbm, zero_hbm, out_hbm,
           acc, idx0, idx1, lidx0, lidx1, rows0, rows1,
           gsem0, gsem1, isem0, isem1, ssem0, ssem1,
           n_chunks, row_base, gather_off, scatter_off, remap_base,
           zrows, ztail, out_row):
    s = lax.axis_index("s")

    pltpu.sync_copy(zero_hbm.at[pl.ds(0, zrows)],
                    acc.at[pl.ds(s * zrows, zrows)])

    @pl.when(s == 0)
    def _():
        pltpu.sync_copy(zero_hbm.at[pl.ds(0, ztail)],
                        acc.at[pl.ds(NT * zrows, ztail)])

    plsc.subcore_barrier()

    idxb = (idx0, idx1)
    lidx = (lidx0, lidx1)
    rows = (rows0, rows1)
    gsem = (gsem0, gsem1)
    isem = (isem0, isem1)
    ssem = (ssem0, ssem1)

    def start_idx(i, b):
        pltpu.async_copy(pidx_hbm.at[row_base + i], idxb[b], isem[b])

    def wait_idx(b):
        pltpu.make_async_copy(pidx_hbm.at[0], idxb[b], isem[b]).wait()

    def start_gather(b):
        pltpu.async_copy(src_hbm.at[idxb[b].at[pl.ds(gather_off, CHUNK)]],
                         rows[b], gsem[b])

    def wait_gather(b):
        pltpu.make_async_copy(src_hbm.at[pl.ds(0, CHUNK)], rows[b],
                              gsem[b]).wait()

    def start_scatter(b):
        pltpu.async_copy(rows[b], acc.at[lidx[b]], ssem[b], add=True)

    def wait_scatter(b):
        pltpu.make_async_copy(rows[b], acc.at[lidx[b]], ssem[b]).wait()

    start_idx(0, 0)
    wait_idx(0)
    start_gather(0)
    start_idx(1, 1)

    def chunk(i, b):
        nb = 1 - b
        wait_gather(b)

        @pl.when(i + 1 < n_chunks)
        def _():
            wait_idx(nb)

            @pl.when(i > 0)
            def _():
                wait_scatter(nb)

            start_gather(nb)

        for j in range(CHUNK // LANES):
            e = idxb[b][pl.ds(scatter_off + j * LANES, LANES)]
            if remap_base is not None:
                l = e - remap_base
                ok = (l >= 0) & (l < HALF_E)
                e = jnp.where(ok, l, HALF_E)
            lidx[b][pl.ds(j * LANES, LANES)] = e

        @pl.when(i + 2 < n_chunks)
        def _():
            start_idx(i + 2, b)

        start_scatter(b)

    def pair(p, carry):
        i = p * 2
        chunk(i, 0)

        @pl.when(i + 1 < n_chunks)
        def _():
            chunk(i + 1, 1)

        return carry

    lax.fori_loop(0, (n_chunks + 1) // 2, pair, 0)

    last = n_chunks - 1
    wait_scatter(1 - (last % 2))
    wait_scatter(last % 2)
    plsc.subcore_barrier()

    pltpu.sync_copy(acc.at[pl.ds(s * zrows, zrows)],
                    out_hbm.at[out_row, pl.ds(s * zrows, zrows)])

    @pl.when(s == 0)
    def _():
        pltpu.sync_copy(acc.at[pl.ds(NT * zrows, ztail)],
                        out_hbm.at[out_row, pl.ds(NT * zrows, ztail)])


def _sc_scratch(n_acc_rows):
    return [
        pltpu.VMEM_SHARED((n_acc_rows, D), jnp.float32),
        pltpu.VMEM((PK,), jnp.int32),
        pltpu.VMEM((PK,), jnp.int32),
        pltpu.VMEM((CHUNK,), jnp.int32),
        pltpu.VMEM((CHUNK,), jnp.int32),
        pltpu.VMEM((CHUNK, D), jnp.float32),
        pltpu.VMEM((CHUNK, D), jnp.float32),
        pltpu.SemaphoreType.DMA,
        pltpu.SemaphoreType.DMA,
        pltpu.SemaphoreType.DMA,
        pltpu.SemaphoreType.DMA,
        pltpu.SemaphoreType.DMA,
        pltpu.SemaphoreType.DMA,
    ]


@functools.partial(
    pl.kernel,
    mesh=_MESH,
    out_type=jax.ShapeDtypeStruct((NC, HALF_E, D), jnp.float32),
    scratch_types=_sc_scratch(ACC_E_ROWS),
)
def _x1_kernel(x0_hbm, pidx_hbm, zero_hbm, x1h_hbm,
               acc, idx0, idx1, lidx0, lidx1, rows0, rows1,
               gs0, gs1, is0, is1, ss0, ss1):
    c = lax.axis_index("c")
    s = lax.axis_index("s")
    _sweep(x0_hbm, pidx_hbm, zero_hbm, x1h_hbm,
           acc, idx0, idx1, lidx0, lidx1, rows0, rows1,
           gs0, gs1, is0, is1, ss0, ss1,
           n_chunks=K1_CHUNKS, row_base=s * K1_CHUNKS,
           gather_off=0, scatter_off=CHUNK, remap_base=c * HALF_E,
           zrows=EZ, ztail=EZ_TAIL, out_row=c)


@functools.partial(
    pl.kernel,
    mesh=_MESH,
    out_type=jax.ShapeDtypeStruct((NC, N_NODES, D), jnp.float32),
    scratch_types=_sc_scratch(N_NODES),
)
def _pre_kernel(x1_hbm, pidx_hbm, zero_hbm, pre_hbm,
                acc, idx0, idx1, lidx0, lidx1, rows0, rows1,
                gs0, gs1, is0, is1, ss0, ss1):
    c = lax.axis_index("c")
    s = lax.axis_index("s")
    _sweep(x1_hbm, pidx_hbm, zero_hbm, pre_hbm,
           acc, idx0, idx1, lidx0, lidx1, rows0, rows1,
           gs0, gs1, is0, is1, ss0, ss1,
           n_chunks=K2_CHUNKS, row_base=(c * NT + s) * K2_CHUNKS,
           gather_off=CHUNK, scatter_off=0, remap_base=None,
           zrows=NZ, ztail=NZ_TAIL, out_row=c)


MM_BLK = 1000


def _mm_body(p0_ref, p1_ref, w_ref, o_ref):
    o_ref[...] = jnp.dot(p0_ref[...] + p1_ref[...], w_ref[...],
                         preferred_element_type=jnp.float32)


def _matmul(p0, p1, w):
    return pl.pallas_call(
        _mm_body,
        grid=(N_NODES // MM_BLK,),
        in_specs=[
            pl.BlockSpec((MM_BLK, D), lambda i: (i, 0)),
            pl.BlockSpec((MM_BLK, D), lambda i: (i, 0)),
            pl.BlockSpec((D, D), lambda i: (0, 0)),
        ],
        out_specs=pl.BlockSpec((MM_BLK, D), lambda i: (i, 0)),
        out_shape=jax.ShapeDtypeStruct((N_NODES, D), jnp.float32),
    )(p0, p1, w)


def kernel(x_0, node_idx, edge_idx, W2):
    pidx = jnp.concatenate([node_idx.reshape(NROWS, CHUNK),
                            edge_idx.reshape(NROWS, CHUNK)], axis=1)
    zero_block = jnp.zeros((EZ, D), jnp.float32)
    x1h = _x1_kernel(x_0, pidx, zero_block)
    x_1 = x1h.reshape(N_HEDGES, D)
    pre = _pre_kernel(x_1, pidx, zero_block)
    x_0_out = _matmul(pre[0], pre[1], W2)
    return (x_0_out, x_1)


# speedup vs baseline: 7.9668x; 1.2718x over previous
import functools

import jax
import jax.numpy as jnp
from jax import lax
from jax.experimental import pallas as pl
from jax.experimental.pallas import tpu as pltpu
from jax.experimental.pallas import tpu_sc as plsc

N_NODES = 10000
N_HEDGES = 20000
NNZ = 320000
D = 128

NC = 2
NT = 16
LANES = 16

HALF_E = N_HEDGES // NC
ACC_E_ROWS = HALF_E + 16
CHUNK = 80
PK = 2 * CHUNK
NROWS = NNZ // CHUNK

EZ = 624
EZ_TAIL = HALF_E - NT * EZ
NZ = 624
NZ_TAIL = N_NODES - NT * NZ

K1_CHUNKS = NROWS // NT
K2_CHUNKS = NROWS // (NC * NT)

_MESH = plsc.VectorSubcoreMesh(core_axis_name="c", subcore_axis_name="s")


NBUF = 4


def _sweep(src_hbm, pidx_hbm, zero_hbm, out_hbm,
           acc, idxb, lidx, rows, gsem, isem, ssem,
           n_chunks, row_base, gather_off, scatter_off, remap_base,
           zrows, ztail, out_row):
    s = lax.axis_index("s")

    pltpu.sync_copy(zero_hbm.at[pl.ds(0, zrows)],
                    acc.at[pl.ds(s * zrows, zrows)])

    @pl.when(s == 0)
    def _():
        pltpu.sync_copy(zero_hbm.at[pl.ds(0, ztail)],
                        acc.at[pl.ds(NT * zrows, ztail)])

    plsc.subcore_barrier()

    def start_idx(i, b):
        pltpu.async_copy(pidx_hbm.at[row_base + i], idxb[b], isem[b])

    def wait_idx(b):
        pltpu.make_async_copy(pidx_hbm.at[0], idxb[b], isem[b]).wait()

    def start_gather(b):
        pltpu.async_copy(src_hbm.at[idxb[b].at[pl.ds(gather_off, CHUNK)]],
                         rows[b], gsem[b])

    def wait_gather(b):
        pltpu.make_async_copy(src_hbm.at[pl.ds(0, CHUNK)], rows[b],
                              gsem[b]).wait()

    def start_scatter(b):
        pltpu.async_copy(rows[b], acc.at[lidx[b]], ssem[b], add=True)

    def wait_scatter(b):
        pltpu.make_async_copy(rows[b], acc.at[lidx[b]], ssem[b]).wait()

    for k in range(NBUF - 1):
        start_idx(k, k)
        wait_idx(k)
        start_gather(k)
    start_idx(NBUF - 1, NBUF - 1)

    def chunk(i, b):
        fb = (b + NBUF - 1) % NBUF
        wait_gather(b)

        @pl.when(i + NBUF - 1 < n_chunks)
        def _():
            wait_idx(fb)

            @pl.when(i > 0)
            def _():
                wait_scatter(fb)

            start_gather(fb)

        for j in range(CHUNK // LANES):
            e = idxb[b][pl.ds(scatter_off + j * LANES, LANES)]
            if remap_base is not None:
                l = e - remap_base
                ok = (l >= 0) & (l < HALF_E)
                e = jnp.where(ok, l, HALF_E)
            lidx[b][pl.ds(j * LANES, LANES)] = e

        @pl.when(i + NBUF < n_chunks)
        def _():
            start_idx(i + NBUF, b)

        start_scatter(b)

    def quad(p, carry):
        i = p * NBUF
        for k in range(NBUF):
            @pl.when(i + k < n_chunks)
            def _(k=k):
                chunk(i + k, k)
        return carry

    lax.fori_loop(0, (n_chunks + NBUF - 1) // NBUF, quad, 0)

    for b in range(NBUF):
        wait_scatter(b)
    plsc.subcore_barrier()

    pltpu.sync_copy(acc.at[pl.ds(s * zrows, zrows)],
                    out_hbm.at[out_row, pl.ds(s * zrows, zrows)])

    @pl.when(s == 0)
    def _():
        pltpu.sync_copy(acc.at[pl.ds(NT * zrows, ztail)],
                        out_hbm.at[out_row, pl.ds(NT * zrows, ztail)])


def _sc_scratch(n_acc_rows):
    return (
        [pltpu.VMEM_SHARED((n_acc_rows, D), jnp.float32)]
        + [pltpu.VMEM((PK,), jnp.int32) for _ in range(NBUF)]
        + [pltpu.VMEM((CHUNK,), jnp.int32) for _ in range(NBUF)]
        + [pltpu.VMEM((CHUNK, D), jnp.float32) for _ in range(NBUF)]
        + [pltpu.SemaphoreType.DMA for _ in range(3 * NBUF)]
    )


@functools.partial(
    pl.kernel,
    mesh=_MESH,
    out_type=jax.ShapeDtypeStruct((NC, HALF_E, D), jnp.float32),
    scratch_types=_sc_scratch(ACC_E_ROWS),
)
def _x1_kernel(x0_hbm, pidx_hbm, zero_hbm, x1h_hbm, acc, *scr):
    c = lax.axis_index("c")
    s = lax.axis_index("s")
    idxb, lidx, rows = scr[:NBUF], scr[NBUF:2*NBUF], scr[2*NBUF:3*NBUF]
    gsem, isem, ssem = (scr[3*NBUF:4*NBUF], scr[4*NBUF:5*NBUF],
                        scr[5*NBUF:6*NBUF])
    _sweep(x0_hbm, pidx_hbm, zero_hbm, x1h_hbm,
           acc, idxb, lidx, rows, gsem, isem, ssem,
           n_chunks=K1_CHUNKS, row_base=s * K1_CHUNKS,
           gather_off=0, scatter_off=CHUNK, remap_base=c * HALF_E,
           zrows=EZ, ztail=EZ_TAIL, out_row=c)


@functools.partial(
    pl.kernel,
    mesh=_MESH,
    out_type=jax.ShapeDtypeStruct((NC, N_NODES, D), jnp.float32),
    scratch_types=_sc_scratch(N_NODES),
)
def _pre_kernel(x1_hbm, pidx_hbm, zero_hbm, pre_hbm, acc, *scr):
    c = lax.axis_index("c")
    s = lax.axis_index("s")
    idxb, lidx, rows = scr[:NBUF], scr[NBUF:2*NBUF], scr[2*NBUF:3*NBUF]
    gsem, isem, ssem = (scr[3*NBUF:4*NBUF], scr[4*NBUF:5*NBUF],
                        scr[5*NBUF:6*NBUF])
    _sweep(x1_hbm, pidx_hbm, zero_hbm, pre_hbm,
           acc, idxb, lidx, rows, gsem, isem, ssem,
           n_chunks=K2_CHUNKS, row_base=(c * NT + s) * K2_CHUNKS,
           gather_off=CHUNK, scatter_off=0, remap_base=None,
           zrows=NZ, ztail=NZ_TAIL, out_row=c)


MM_BLK = 1000


def _mm_body(p0_ref, p1_ref, w_ref, o_ref):
    o_ref[...] = jnp.dot(p0_ref[...] + p1_ref[...], w_ref[...],
                         preferred_element_type=jnp.float32)


def _matmul(p0, p1, w):
    return pl.pallas_call(
        _mm_body,
        grid=(N_NODES // MM_BLK,),
        in_specs=[
            pl.BlockSpec((MM_BLK, D), lambda i: (i, 0)),
            pl.BlockSpec((MM_BLK, D), lambda i: (i, 0)),
            pl.BlockSpec((D, D), lambda i: (0, 0)),
        ],
        out_specs=pl.BlockSpec((MM_BLK, D), lambda i: (i, 0)),
        out_shape=jax.ShapeDtypeStruct((N_NODES, D), jnp.float32),
    )(p0, p1, w)


def kernel(x_0, node_idx, edge_idx, W2):
    pidx = jnp.concatenate([node_idx.reshape(NROWS, CHUNK),
                            edge_idx.reshape(NROWS, CHUNK)], axis=1)
    zero_block = jnp.zeros((EZ, D), jnp.float32)
    x1h = _x1_kernel(x_0, pidx, zero_block)
    x_1 = x1h.reshape(N_HEDGES, D)
    pre = _pre_kernel(x_1, pidx, zero_block)
    x_0_out = _matmul(pre[0], pre[1], W2)
    return (x_0_out, x_1)


# speedup vs baseline: 8.1533x; 1.0234x over previous
import functools

import jax
import jax.numpy as jnp
from jax import lax
from jax.experimental import pallas as pl
from jax.experimental.pallas import tpu as pltpu
from jax.experimental.pallas import tpu_sc as plsc

N_NODES = 10000
N_HEDGES = 20000
NNZ = 320000
D = 128

NC = 2
NT = 16
LANES = 16

HALF_E = N_HEDGES // NC
ACC_E_ROWS = HALF_E + 16
CHUNK = 80
PK = 2 * CHUNK
NROWS = NNZ // CHUNK

EZ = 624
EZ_TAIL = HALF_E - NT * EZ
NZ = 624
NZ_TAIL = N_NODES - NT * NZ

K1_CHUNKS = NROWS // NT
K2_CHUNKS = NROWS // (NC * NT)

_MESH = plsc.VectorSubcoreMesh(core_axis_name="c", subcore_axis_name="s")


NBUF = 4


def _sweep(src_hbm, pidx_hbm, out_hbm,
           acc, idxb, lidx, rows, gsem, isem, ssem,
           n_chunks, row_base, gather_off, scatter_off, remap_base,
           zrows, ztail, out_row):
    s = lax.axis_index("s")

    def zrow(r, carry):
        for j in range(D // LANES):
            rows[0][r, pl.ds(j * LANES, LANES)] = jnp.zeros((LANES,),
                                                            jnp.float32)
        return carry
    lax.fori_loop(0, CHUNK, zrow, 0)
    for k in range(zrows // CHUNK):
        pltpu.sync_copy(rows[0], acc.at[pl.ds(s * zrows + k * CHUNK, CHUNK)])
    rem = zrows - (zrows // CHUNK) * CHUNK
    if rem:
        pltpu.sync_copy(rows[0].at[pl.ds(0, rem)],
                        acc.at[pl.ds(s * zrows + zrows - rem, rem)])

    @pl.when(s == 0)
    def _():
        pltpu.sync_copy(rows[0].at[pl.ds(0, ztail)],
                        acc.at[pl.ds(NT * zrows, ztail)])

    plsc.subcore_barrier()

    def start_idx(i, b):
        pltpu.async_copy(pidx_hbm.at[row_base + i], idxb[b], isem[b])

    def wait_idx(b):
        pltpu.make_async_copy(pidx_hbm.at[0], idxb[b], isem[b]).wait()

    def start_gather(b):
        pltpu.async_copy(src_hbm.at[idxb[b].at[pl.ds(gather_off, CHUNK)]],
                         rows[b], gsem[b])

    def wait_gather(b):
        pltpu.make_async_copy(src_hbm.at[pl.ds(0, CHUNK)], rows[b],
                              gsem[b]).wait()

    def start_scatter(b):
        pltpu.async_copy(rows[b], acc.at[lidx[b]], ssem[b], add=True)

    def wait_scatter(b):
        pltpu.make_async_copy(rows[b], acc.at[lidx[b]], ssem[b]).wait()

    for k in range(NBUF - 1):
        start_idx(k, k)
        wait_idx(k)
        start_gather(k)
    start_idx(NBUF - 1, NBUF - 1)

    def chunk(i, b):
        fb = (b + NBUF - 1) % NBUF
        wait_gather(b)

        @pl.when(i + NBUF - 1 < n_chunks)
        def _():
            wait_idx(fb)

            @pl.when(i > 0)
            def _():
                wait_scatter(fb)

            start_gather(fb)

        for j in range(CHUNK // LANES):
            e = idxb[b][pl.ds(scatter_off + j * LANES, LANES)]
            if remap_base is not None:
                l = e - remap_base
                ok = (l >= 0) & (l < HALF_E)
                e = jnp.where(ok, l, HALF_E)
            lidx[b][pl.ds(j * LANES, LANES)] = e

        @pl.when(i + NBUF < n_chunks)
        def _():
            start_idx(i + NBUF, b)

        start_scatter(b)

    def quad(p, carry):
        i = p * NBUF
        for k in range(NBUF):
            @pl.when(i + k < n_chunks)
            def _(k=k):
                chunk(i + k, k)
        return carry

    lax.fori_loop(0, (n_chunks + NBUF - 1) // NBUF, quad, 0)

    for b in range(NBUF):
        wait_scatter(b)
    plsc.subcore_barrier()

    pltpu.sync_copy(acc.at[pl.ds(s * zrows, zrows)],
                    out_hbm.at[out_row, pl.ds(s * zrows, zrows)])

    @pl.when(s == 0)
    def _():
        pltpu.sync_copy(acc.at[pl.ds(NT * zrows, ztail)],
                        out_hbm.at[out_row, pl.ds(NT * zrows, ztail)])


def _sc_scratch(n_acc_rows):
    return (
        [pltpu.VMEM_SHARED((n_acc_rows, D), jnp.float32)]
        + [pltpu.VMEM((PK,), jnp.int32) for _ in range(NBUF)]
        + [pltpu.VMEM((CHUNK,), jnp.int32) for _ in range(NBUF)]
        + [pltpu.VMEM((CHUNK, D), jnp.float32) for _ in range(NBUF)]
        + [pltpu.SemaphoreType.DMA for _ in range(3 * NBUF)]
    )


@functools.partial(
    pl.kernel,
    mesh=_MESH,
    out_type=jax.ShapeDtypeStruct((NC, HALF_E, D), jnp.float32),
    scratch_types=_sc_scratch(ACC_E_ROWS),
)
def _x1_kernel(x0_hbm, pidx_hbm, x1h_hbm, acc, *scr):
    c = lax.axis_index("c")
    s = lax.axis_index("s")
    idxb, lidx, rows = scr[:NBUF], scr[NBUF:2*NBUF], scr[2*NBUF:3*NBUF]
    gsem, isem, ssem = (scr[3*NBUF:4*NBUF], scr[4*NBUF:5*NBUF],
                        scr[5*NBUF:6*NBUF])
    _sweep(x0_hbm, pidx_hbm, x1h_hbm,
           acc, idxb, lidx, rows, gsem, isem, ssem,
           n_chunks=K1_CHUNKS, row_base=s * K1_CHUNKS,
           gather_off=0, scatter_off=CHUNK, remap_base=c * HALF_E,
           zrows=EZ, ztail=EZ_TAIL, out_row=c)


@functools.partial(
    pl.kernel,
    mesh=_MESH,
    out_type=jax.ShapeDtypeStruct((NC, N_NODES, D), jnp.float32),
    scratch_types=_sc_scratch(N_NODES),
)
def _pre_kernel(x1_hbm, pidx_hbm, pre_hbm, acc, *scr):
    c = lax.axis_index("c")
    s = lax.axis_index("s")
    idxb, lidx, rows = scr[:NBUF], scr[NBUF:2*NBUF], scr[2*NBUF:3*NBUF]
    gsem, isem, ssem = (scr[3*NBUF:4*NBUF], scr[4*NBUF:5*NBUF],
                        scr[5*NBUF:6*NBUF])
    _sweep(x1_hbm, pidx_hbm, pre_hbm,
           acc, idxb, lidx, rows, gsem, isem, ssem,
           n_chunks=K2_CHUNKS, row_base=(c * NT + s) * K2_CHUNKS,
           gather_off=CHUNK, scatter_off=0, remap_base=None,
           zrows=NZ, ztail=NZ_TAIL, out_row=c)


MM_BLK = 1000


def _mm_body(p0_ref, p1_ref, w_ref, o_ref):
    o_ref[...] = jnp.dot(p0_ref[...] + p1_ref[...], w_ref[...],
                         preferred_element_type=jnp.float32)


def _matmul(p0, p1, w):
    return pl.pallas_call(
        _mm_body,
        grid=(N_NODES // MM_BLK,),
        in_specs=[
            pl.BlockSpec((MM_BLK, D), lambda i: (i, 0)),
            pl.BlockSpec((MM_BLK, D), lambda i: (i, 0)),
            pl.BlockSpec((D, D), lambda i: (0, 0)),
        ],
        out_specs=pl.BlockSpec((MM_BLK, D), lambda i: (i, 0)),
        out_shape=jax.ShapeDtypeStruct((N_NODES, D), jnp.float32),
    )(p0, p1, w)


def kernel(x_0, node_idx, edge_idx, W2):
    pidx = jnp.concatenate([node_idx.reshape(NROWS, CHUNK),
                            edge_idx.reshape(NROWS, CHUNK)], axis=1)
    x1h = _x1_kernel(x_0, pidx)
    x_1 = x1h.reshape(N_HEDGES, D)
    pre = _pre_kernel(x_1, pidx)
    x_0_out = _matmul(pre[0], pre[1], W2)
    return (x_0_out, x_1)


# speedup vs baseline: 8.2411x; 1.0108x over previous
import functools

import jax
import jax.numpy as jnp
from jax import lax
from jax.experimental import pallas as pl
from jax.experimental.pallas import tpu as pltpu
from jax.experimental.pallas import tpu_sc as plsc

N_NODES = 10000
N_HEDGES = 20000
NNZ = 320000
D = 128

NC = 2
NT = 16
LANES = 16

HALF_E = N_HEDGES // NC
ACC_E_ROWS = HALF_E + 16
CHUNK = 80
PK = 2 * CHUNK
NROWS = NNZ // CHUNK

EZ = 624
EZ_TAIL = HALF_E - NT * EZ
NZ = 624
NZ_TAIL = N_NODES - NT * NZ

K1_CHUNKS = NROWS // NT
K2_CHUNKS = NROWS // (NC * NT)


def _split_scr(scr):
    o = 0
    idxb = scr[o:o + NIDX]; o += NIDX
    lidx = scr[o:o + NBUF]; o += NBUF
    rows = scr[o:o + NBUF]; o += NBUF
    isem = scr[o:o + NIDX]; o += NIDX
    gsem = scr[o:o + NBUF]; o += NBUF
    ssem = scr[o:o + NBUF]; o += NBUF
    return idxb, lidx, rows, gsem, isem, ssem

_MESH = plsc.VectorSubcoreMesh(core_axis_name="c", subcore_axis_name="s")


NBUF = 4
NIDX = 8


def _sweep(src_hbm, pidx_hbm, out_hbm,
           acc, idxb, lidx, rows, gsem, isem, ssem,
           n_chunks, row_base, gather_off, scatter_off, remap_base,
           zrows, ztail, out_row):
    s = lax.axis_index("s")

    def zrow(r, carry):
        for j in range(D // LANES):
            rows[0][r, pl.ds(j * LANES, LANES)] = jnp.zeros((LANES,),
                                                            jnp.float32)
        return carry
    lax.fori_loop(0, CHUNK, zrow, 0)
    for k in range(zrows // CHUNK):
        pltpu.sync_copy(rows[0], acc.at[pl.ds(s * zrows + k * CHUNK, CHUNK)])
    rem = zrows - (zrows // CHUNK) * CHUNK
    if rem:
        pltpu.sync_copy(rows[0].at[pl.ds(0, rem)],
                        acc.at[pl.ds(s * zrows + zrows - rem, rem)])

    @pl.when(s == 0)
    def _():
        pltpu.sync_copy(rows[0].at[pl.ds(0, ztail)],
                        acc.at[pl.ds(NT * zrows, ztail)])

    plsc.subcore_barrier()

    def start_idx(i, ib):
        pltpu.async_copy(pidx_hbm.at[row_base + i], idxb[ib], isem[ib])

    def wait_idx(ib):
        pltpu.make_async_copy(pidx_hbm.at[0], idxb[ib], isem[ib]).wait()

    def start_gather(b, ib):
        pltpu.async_copy(src_hbm.at[idxb[ib].at[pl.ds(gather_off, CHUNK)]],
                         rows[b], gsem[b])

    def wait_gather(b):
        pltpu.make_async_copy(src_hbm.at[pl.ds(0, CHUNK)], rows[b],
                              gsem[b]).wait()

    def start_scatter(b):
        pltpu.async_copy(rows[b], acc.at[lidx[b]], ssem[b], add=True)

    def wait_scatter(b):
        pltpu.make_async_copy(rows[b], acc.at[lidx[b]], ssem[b]).wait()

    for k in range(NBUF - 1):
        start_idx(k, k)
        wait_idx(k)
        start_gather(k, k)
    for k in range(NBUF - 1, NIDX):
        start_idx(k, k)

    def chunk(i, b, ib):
        fb = (b + NBUF - 1) % NBUF
        fi = (ib + NBUF - 1) % NIDX
        wait_gather(b)

        @pl.when(i + NBUF - 1 < n_chunks)
        def _():
            wait_idx(fi)

            @pl.when(i > 0)
            def _():
                wait_scatter(fb)

            start_gather(fb, fi)

        for j in range(CHUNK // LANES):
            e = idxb[ib][pl.ds(scatter_off + j * LANES, LANES)]
            if remap_base is not None:
                l = e - remap_base
                ok = (l >= 0) & (l < HALF_E)
                e = jnp.where(ok, l, HALF_E)
            lidx[b][pl.ds(j * LANES, LANES)] = e

        @pl.when(i + NIDX < n_chunks)
        def _():
            start_idx(i + NIDX, ib)

        start_scatter(b)

    def oct_(p, carry):
        i = p * NIDX
        for k in range(NIDX):
            @pl.when(i + k < n_chunks)
            def _(k=k):
                chunk(i + k, k % NBUF, k)
        return carry

    lax.fori_loop(0, (n_chunks + NIDX - 1) // NIDX, oct_, 0)

    for b in range(NBUF):
        wait_scatter(b)
    plsc.subcore_barrier()

    pltpu.sync_copy(acc.at[pl.ds(s * zrows, zrows)],
                    out_hbm.at[out_row, pl.ds(s * zrows, zrows)])

    @pl.when(s == 0)
    def _():
        pltpu.sync_copy(acc.at[pl.ds(NT * zrows, ztail)],
                        out_hbm.at[out_row, pl.ds(NT * zrows, ztail)])


def _sc_scratch(n_acc_rows):
    return (
        [pltpu.VMEM_SHARED((n_acc_rows, D), jnp.float32)]
        + [pltpu.VMEM((PK,), jnp.int32) for _ in range(NIDX)]
        + [pltpu.VMEM((CHUNK,), jnp.int32) for _ in range(NBUF)]
        + [pltpu.VMEM((CHUNK, D), jnp.float32) for _ in range(NBUF)]
        + [pltpu.SemaphoreType.DMA for _ in range(NIDX + 2 * NBUF)]
    )


@functools.partial(
    pl.kernel,
    mesh=_MESH,
    out_type=jax.ShapeDtypeStruct((NC, HALF_E, D), jnp.float32),
    scratch_types=_sc_scratch(ACC_E_ROWS),
)
def _x1_kernel(x0_hbm, pidx_hbm, x1h_hbm, acc, *scr):
    c = lax.axis_index("c")
    s = lax.axis_index("s")
    idxb, lidx, rows, gsem, isem, ssem = _split_scr(scr)
    _sweep(x0_hbm, pidx_hbm, x1h_hbm,
           acc, idxb, lidx, rows, gsem, isem, ssem,
           n_chunks=K1_CHUNKS, row_base=s * K1_CHUNKS,
           gather_off=0, scatter_off=CHUNK, remap_base=c * HALF_E,
           zrows=EZ, ztail=EZ_TAIL, out_row=c)


@functools.partial(
    pl.kernel,
    mesh=_MESH,
    out_type=jax.ShapeDtypeStruct((NC, N_NODES, D), jnp.float32),
    scratch_types=_sc_scratch(N_NODES),
)
def _pre_kernel(x1_hbm, pidx_hbm, pre_hbm, acc, *scr):
    c = lax.axis_index("c")
    s = lax.axis_index("s")
    idxb, lidx, rows, gsem, isem, ssem = _split_scr(scr)
    _sweep(x1_hbm, pidx_hbm, pre_hbm,
           acc, idxb, lidx, rows, gsem, isem, ssem,
           n_chunks=K2_CHUNKS, row_base=(c * NT + s) * K2_CHUNKS,
           gather_off=CHUNK, scatter_off=0, remap_base=None,
           zrows=NZ, ztail=NZ_TAIL, out_row=c)


MM_BLK = 1000


def _mm_body(p0_ref, p1_ref, w_ref, o_ref):
    o_ref[...] = jnp.dot(p0_ref[...] + p1_ref[...], w_ref[...],
                         preferred_element_type=jnp.float32)


def _matmul(p0, p1, w):
    return pl.pallas_call(
        _mm_body,
        grid=(N_NODES // MM_BLK,),
        in_specs=[
            pl.BlockSpec((MM_BLK, D), lambda i: (i, 0)),
            pl.BlockSpec((MM_BLK, D), lambda i: (i, 0)),
            pl.BlockSpec((D, D), lambda i: (0, 0)),
        ],
        out_specs=pl.BlockSpec((MM_BLK, D), lambda i: (i, 0)),
        out_shape=jax.ShapeDtypeStruct((N_NODES, D), jnp.float32),
    )(p0, p1, w)


def kernel(x_0, node_idx, edge_idx, W2):
    pidx = jnp.concatenate([node_idx.reshape(NROWS, CHUNK),
                            edge_idx.reshape(NROWS, CHUNK)], axis=1)
    x1h = _x1_kernel(x_0, pidx)
    x_1 = x1h.reshape(N_HEDGES, D)
    pre = _pre_kernel(x_1, pidx)
    x_0_out = _matmul(pre[0], pre[1], W2)
    return (x_0_out, x_1)
